# baseline (speedup 1.0000x reference)
import jax
import jax.numpy as jnp
from jax import lax
from jax.experimental import pallas as pl
from jax.experimental.pallas import tpu as pltpu, tpu_sc as plsc

_NC = 2
_NS = 16
_NW = _NC * _NS

_N = 2 * 256 * 256
_H = 4
_D = 64
_W = _H * _D
_RW = _N // _NW
_CH = 64
_NSTEP = _RW // (2 * _CH)


def _sc_body(tabk_hbm, tabv_hbm, idx_hbm, outk_hbm, outv_hbm,
             idx_v, bk0, bv0, bk1, bv1, gs0, gs1, ws0, ws1):
    wid = lax.axis_index("s") * _NC + lax.axis_index("c")
    base = wid * _RW
    pltpu.sync_copy(idx_hbm.at[pl.ds(base, _RW)], idx_v)

    def step(t, carry):
        oa = 2 * t * _CH
        ob = oa + _CH
        rowa = base + oa
        rowb = base + ob
        ia = idx_v.at[pl.ds(oa, _CH)]
        ib = idx_v.at[pl.ds(ob, _CH)]

        @pl.when(t != 0)
        def _():
            pltpu.make_async_copy(bk0, outk_hbm.at[pl.ds(rowa, _CH)], ws0).wait()
            pltpu.make_async_copy(bv0, outv_hbm.at[pl.ds(rowa, _CH)], ws0).wait()

        pltpu.async_copy(tabk_hbm.at[ia], bk0, gs0)
        pltpu.async_copy(tabv_hbm.at[ia], bv0, gs0)

        @pl.when(t != 0)
        def _():
            pltpu.make_async_copy(bk1, outk_hbm.at[pl.ds(rowb, _CH)], ws1).wait()
            pltpu.make_async_copy(bv1, outv_hbm.at[pl.ds(rowb, _CH)], ws1).wait()

        pltpu.async_copy(tabk_hbm.at[ib], bk1, gs1)
        pltpu.async_copy(tabv_hbm.at[ib], bv1, gs1)

        pltpu.make_async_copy(tabk_hbm.at[ia], bk0, gs0).wait()
        pltpu.make_async_copy(tabv_hbm.at[ia], bv0, gs0).wait()
        pltpu.async_copy(bk0, outk_hbm.at[pl.ds(rowa, _CH)], ws0)
        pltpu.async_copy(bv0, outv_hbm.at[pl.ds(rowa, _CH)], ws0)

        pltpu.make_async_copy(tabk_hbm.at[ib], bk1, gs1).wait()
        pltpu.make_async_copy(tabv_hbm.at[ib], bv1, gs1).wait()
        pltpu.async_copy(bk1, outk_hbm.at[pl.ds(rowb, _CH)], ws1)
        pltpu.async_copy(bv1, outv_hbm.at[pl.ds(rowb, _CH)], ws1)
        return carry

    lax.fori_loop(0, _NSTEP, step, 0)

    pltpu.make_async_copy(bk0, outk_hbm.at[pl.ds(base, _CH)], ws0).wait()
    pltpu.make_async_copy(bv0, outv_hbm.at[pl.ds(base, _CH)], ws0).wait()
    pltpu.make_async_copy(bk1, outk_hbm.at[pl.ds(base, _CH)], ws1).wait()
    pltpu.make_async_copy(bv1, outv_hbm.at[pl.ds(base, _CH)], ws1).wait()


def kernel(inputs, brother_table, relation_type, num_heads):
    del relation_type, num_heads
    idx = inputs.reshape(-1).astype(jnp.int32)
    tab_k4 = jnp.tile(brother_table[:, :_D], (1, _H)).reshape(-1, _H, _D)
    tab_v4 = jnp.tile(brother_table[:, _D:], (1, _H)).reshape(-1, _H, _D)

    mesh = plsc.VectorSubcoreMesh(core_axis_name="c", subcore_axis_name="s")
    f = pl.kernel(
        _sc_body,
        out_type=(
            jax.ShapeDtypeStruct((_N, _H, _D), jnp.float32),
            jax.ShapeDtypeStruct((_N, _H, _D), jnp.float32),
        ),
        mesh=mesh,
        compiler_params=pltpu.CompilerParams(use_tc_tiling_on_sc=False),
        scratch_types=[
            pltpu.VMEM((_RW,), jnp.int32),
            pltpu.VMEM((_CH, _H, _D), jnp.float32),
            pltpu.VMEM((_CH, _H, _D), jnp.float32),
            pltpu.VMEM((_CH, _H, _D), jnp.float32),
            pltpu.VMEM((_CH, _H, _D), jnp.float32),
            pltpu.SemaphoreType.DMA,
            pltpu.SemaphoreType.DMA,
            pltpu.SemaphoreType.DMA,
            pltpu.SemaphoreType.DMA,
        ],
    )
    outk, outv = f(tab_k4, tab_v4, idx)
    out_shape = (inputs.shape[0] * _H, inputs.shape[1], inputs.shape[2], _D)
    return outk.reshape(out_shape), outv.reshape(out_shape)


# speedup vs baseline: 2.3543x; 2.3543x over previous
import functools
import jax
import jax.numpy as jnp
from jax.experimental import pallas as pl

_B = 2
_S = 256
_H = 4
_D = 64
_V = 130
_TI = 8


def _tc_body(tk0_ref, tk1_ref, tv0_ref, tv1_ref, ide_ref, outk_ref, outv_ref):
    tk0 = tk0_ref[...]
    tk1 = tk1_ref[...]
    tv0 = tv0_ref[...]
    tv1 = tv1_ref[...]
    for t in range(_TI):
        ids = jnp.broadcast_to(ide_ref[0, t], (_D, _H * _D))
        low = ids < 128
        i0 = jnp.where(low, ids, 0)
        i1 = jnp.where(low, 0, ids - 128)
        outk_ref[0, t] = jnp.where(
            low,
            jnp.take_along_axis(tk0, i0, axis=1),
            jnp.take_along_axis(tk1, i1, axis=1),
        )
        outv_ref[0, t] = jnp.where(
            low,
            jnp.take_along_axis(tv0, i0, axis=1),
            jnp.take_along_axis(tv1, i1, axis=1),
        )


def kernel(inputs, brother_table, relation_type, num_heads):
    del relation_type, num_heads
    tk = brother_table[:, :_D].T
    tv = brother_table[:, _D:].T
    pad = ((0, 0), (0, 256 - _V))
    tkp = jnp.pad(tk, pad)
    tvp = jnp.pad(tv, pad)
    tk0, tk1 = tkp[:, :128], tkp[:, 128:]
    tv0, tv1 = tvp[:, :128], tvp[:, 128:]

    ide = jnp.broadcast_to(
        inputs.reshape(_B, _S, _S, 1), (_B, _S, _S, _H)
    ).reshape(_B * _S * _S // (_TI * _D), _TI, 1, _H * _D)

    grid = (_B * _S * _S // (_TI * _D),)

    def ide_map(g):
        return (g, 0, 0, 0)

    def out_map(g):
        return (g // 32, g % 32, 0, 0)

    f = pl.pallas_call(
        _tc_body,
        grid=grid,
        in_specs=[
            pl.BlockSpec((_D, 128), lambda g: (0, 0)),
            pl.BlockSpec((_D, 128), lambda g: (0, 0)),
            pl.BlockSpec((_D, 128), lambda g: (0, 0)),
            pl.BlockSpec((_D, 128), lambda g: (0, 0)),
            pl.BlockSpec((1, _TI, 1, _H * _D), ide_map),
        ],
        out_specs=[
            pl.BlockSpec((1, _TI, _D, _H * _D), out_map),
            pl.BlockSpec((1, _TI, _D, _H * _D), out_map),
        ],
        out_shape=[
            jax.ShapeDtypeStruct((_B * _H, _S, _D, _S), jnp.float32),
            jax.ShapeDtypeStruct((_B * _H, _S, _D, _S), jnp.float32),
        ],
    )
    outk, outv = f(tk0, tk1, tv0, tv1, ide)
    return (
        jnp.transpose(outk, (0, 1, 3, 2)),
        jnp.transpose(outv, (0, 1, 3, 2)),
    )


# speedup vs baseline: 3.0134x; 1.2800x over previous
import functools
import jax
import jax.numpy as jnp
from jax.experimental import pallas as pl

_B = 2
_S = 256
_H = 4
_D = 64
_V = 130
_TI = 8


def _tc_body(tk2_ref, tv2_ref, tk1_ref, tv1_ref, ide_ref, outk_ref, outv_ref):
    tk2 = tk2_ref[...]
    tv2 = tv2_ref[...]
    tk1 = jnp.broadcast_to(tk1_ref[...], (_D, _H * _D))
    tv1 = jnp.broadcast_to(tv1_ref[...], (_D, _H * _D))
    zero = jnp.zeros((_D, _H * _D), jnp.float32)
    for t in range(_TI):
        ids = jnp.broadcast_to(ide_ref[0, t], (_D, _H * _D))
        i2 = jnp.maximum(ids - 2, 0)
        big = ids >= 2
        one = ids == 1
        gk = jnp.take_along_axis(tk2, i2, axis=1)
        gv = jnp.take_along_axis(tv2, i2, axis=1)
        outk_ref[0, t] = jnp.where(big, gk, jnp.where(one, tk1, zero))
        outv_ref[0, t] = jnp.where(big, gv, jnp.where(one, tv1, zero))


def kernel(inputs, brother_table, relation_type, num_heads):
    del relation_type, num_heads
    tk = brother_table[:, :_D].T
    tv = brother_table[:, _D:].T
    tk2, tv2 = tk[:, 2:], tv[:, 2:]
    tk1, tv1 = tk[:, 1:2], tv[:, 1:2]

    ide = jnp.broadcast_to(
        inputs.reshape(_B, _S, _S, 1), (_B, _S, _S, _H)
    ).reshape(_B * _S * _S // (_TI * _D), _TI, 1, _H * _D)

    grid = (_B * _S * _S // (_TI * _D),)

    def ide_map(g):
        return (g, 0, 0, 0)

    def out_map(g):
        return (g // 32, g % 32, 0, 0)

    f = pl.pallas_call(
        _tc_body,
        grid=grid,
        in_specs=[
            pl.BlockSpec((_D, 128), lambda g: (0, 0)),
            pl.BlockSpec((_D, 128), lambda g: (0, 0)),
            pl.BlockSpec((_D, 1), lambda g: (0, 0)),
            pl.BlockSpec((_D, 1), lambda g: (0, 0)),
            pl.BlockSpec((1, _TI, 1, _H * _D), ide_map),
        ],
        out_specs=[
            pl.BlockSpec((1, _TI, _D, _H * _D), out_map),
            pl.BlockSpec((1, _TI, _D, _H * _D), out_map),
        ],
        out_shape=[
            jax.ShapeDtypeStruct((_B * _H, _S, _D, _S), jnp.float32),
            jax.ShapeDtypeStruct((_B * _H, _S, _D, _S), jnp.float32),
        ],
    )
    outk, outv = f(tk2, tv2, tk1, tv1, ide)
    return (
        jnp.transpose(outk, (0, 1, 3, 2)),
        jnp.transpose(outv, (0, 1, 3, 2)),
    )


# speedup vs baseline: 3.7112x; 1.2316x over previous
import functools
import jax
import jax.numpy as jnp
from jax.experimental import pallas as pl

_B = 2
_S = 256
_H = 4
_D = 64
_V = 130
_TI = 16


def _tc_body(tk2_ref, tv2_ref, tk1_ref, tv1_ref, ide_ref, outk_ref, outv_ref):
    tk2 = tk2_ref[...]
    tv2 = tv2_ref[...]
    tk1 = jnp.broadcast_to(tk1_ref[...], (_D, _H * _D))
    tv1 = jnp.broadcast_to(tv1_ref[...], (_D, _H * _D))
    zero = jnp.zeros((_D, _H * _D), jnp.float32)
    for t in range(_TI):
        ids_row = ide_ref[0, t]
        i2 = jnp.broadcast_to(jnp.maximum(ids_row - 2, 0), (_D, _H * _D))
        big = jnp.broadcast_to(ids_row >= 2, (_D, _H * _D))
        one = jnp.broadcast_to(ids_row == 1, (_D, _H * _D))
        gk = jnp.take_along_axis(tk2, i2, axis=1)
        gv = jnp.take_along_axis(tv2, i2, axis=1)
        outk_ref[0, t] = jnp.where(big, gk, jnp.where(one, tk1, zero))
        outv_ref[0, t] = jnp.where(big, gv, jnp.where(one, tv1, zero))


def kernel(inputs, brother_table, relation_type, num_heads):
    del relation_type, num_heads
    tk = brother_table[:, :_D].T
    tv = brother_table[:, _D:].T
    tk2, tv2 = tk[:, 2:], tv[:, 2:]
    tk1, tv1 = tk[:, 1:2], tv[:, 1:2]

    ide = jnp.broadcast_to(
        inputs.reshape(_B, _S, _S, 1), (_B, _S, _S, _H)
    ).reshape(_B * _S * _S // (_TI * _D), _TI, 1, _H * _D)

    grid = (_B * _S * _S // (_TI * _D),)

    def ide_map(g):
        return (g, 0, 0, 0)

    def out_map(g):
        return (g // (_S // _TI), g % (_S // _TI), 0, 0)

    f = pl.pallas_call(
        _tc_body,
        grid=grid,
        in_specs=[
            pl.BlockSpec((_D, 128), lambda g: (0, 0)),
            pl.BlockSpec((_D, 128), lambda g: (0, 0)),
            pl.BlockSpec((_D, 1), lambda g: (0, 0)),
            pl.BlockSpec((_D, 1), lambda g: (0, 0)),
            pl.BlockSpec((1, _TI, 1, _H * _D), ide_map),
        ],
        out_specs=[
            pl.BlockSpec((1, _TI, _D, _H * _D), out_map),
            pl.BlockSpec((1, _TI, _D, _H * _D), out_map),
        ],
        out_shape=[
            jax.ShapeDtypeStruct((_B * _H, _S, _D, _S), jnp.float32),
            jax.ShapeDtypeStruct((_B * _H, _S, _D, _S), jnp.float32),
        ],
    )
    outk, outv = f(tk2, tv2, tk1, tv1, ide)
    return (
        jnp.transpose(outk, (0, 1, 3, 2)),
        jnp.transpose(outv, (0, 1, 3, 2)),
    )


# speedup vs baseline: 3.9223x; 1.0569x over previous
import functools
import jax
import jax.numpy as jnp
from jax.experimental import pallas as pl

_B = 2
_S = 256
_H = 4
_D = 64
_V = 130
_TI = 32


def _tc_body(tk2_ref, tv2_ref, tk1_ref, tv1_ref, ide_ref, outk_ref, outv_ref):
    tk2 = tk2_ref[...]
    tv2 = tv2_ref[...]
    tk1 = jnp.broadcast_to(tk1_ref[...], (_D, _H * _D))
    tv1 = jnp.broadcast_to(tv1_ref[...], (_D, _H * _D))
    zero = jnp.zeros((_D, _H * _D), jnp.float32)
    for t in range(_TI):
        ids_row = ide_ref[0, t]
        i2 = jnp.broadcast_to(jnp.maximum(ids_row - 2, 0), (_D, _H * _D))
        big = jnp.broadcast_to(ids_row >= 2, (_D, _H * _D))
        one = jnp.broadcast_to(ids_row == 1, (_D, _H * _D))
        gk = jnp.take_along_axis(tk2, i2, axis=1)
        gv = jnp.take_along_axis(tv2, i2, axis=1)
        outk_ref[0, t] = jnp.where(big, gk, jnp.where(one, tk1, zero))
        outv_ref[0, t] = jnp.where(big, gv, jnp.where(one, tv1, zero))


def kernel(inputs, brother_table, relation_type, num_heads):
    del relation_type, num_heads
    tk = brother_table[:, :_D].T
    tv = brother_table[:, _D:].T
    tk2, tv2 = tk[:, 2:], tv[:, 2:]
    tk1, tv1 = tk[:, 1:2], tv[:, 1:2]

    ide = jnp.broadcast_to(
        inputs.reshape(_B, _S, _S, 1), (_B, _S, _S, _H)
    ).reshape(_B * _S * _S // (_TI * _D), _TI, 1, _H * _D)

    grid = (_B * _S * _S // (_TI * _D),)

    def ide_map(g):
        return (g, 0, 0, 0)

    def out_map(g):
        return (g // (_S // _TI), g % (_S // _TI), 0, 0)

    f = pl.pallas_call(
        _tc_body,
        grid=grid,
        in_specs=[
            pl.BlockSpec((_D, 128), lambda g: (0, 0)),
            pl.BlockSpec((_D, 128), lambda g: (0, 0)),
            pl.BlockSpec((_D, 1), lambda g: (0, 0)),
            pl.BlockSpec((_D, 1), lambda g: (0, 0)),
            pl.BlockSpec((1, _TI, 1, _H * _D), ide_map),
        ],
        out_specs=[
            pl.BlockSpec((1, _TI, _D, _H * _D), out_map),
            pl.BlockSpec((1, _TI, _D, _H * _D), out_map),
        ],
        out_shape=[
            jax.ShapeDtypeStruct((_B * _H, _S, _D, _S), jnp.float32),
            jax.ShapeDtypeStruct((_B * _H, _S, _D, _S), jnp.float32),
        ],
    )
    outk, outv = f(tk2, tv2, tk1, tv1, ide)
    return (
        jnp.transpose(outk, (0, 1, 3, 2)),
        jnp.transpose(outv, (0, 1, 3, 2)),
    )


# speedup vs baseline: 3.9600x; 1.0096x over previous
import functools
import jax
import jax.numpy as jnp
from jax.experimental import pallas as pl

_B = 2
_S = 256
_H = 4
_D = 64
_V = 130
_TI = 64


def _tc_body(tk2_ref, tv2_ref, tk1_ref, tv1_ref, ide_ref, outk_ref, outv_ref):
    tk2 = tk2_ref[...]
    tv2 = tv2_ref[...]
    tk1 = jnp.broadcast_to(tk1_ref[...], (_D, _H * _D))
    tv1 = jnp.broadcast_to(tv1_ref[...], (_D, _H * _D))
    zero = jnp.zeros((_D, _H * _D), jnp.float32)
    for t in range(_TI):
        ids_row = ide_ref[0, t]
        i2 = jnp.broadcast_to(jnp.maximum(ids_row - 2, 0), (_D, _H * _D))
        big = jnp.broadcast_to(ids_row >= 2, (_D, _H * _D))
        one = jnp.broadcast_to(ids_row == 1, (_D, _H * _D))
        gk = jnp.take_along_axis(tk2, i2, axis=1)
        gv = jnp.take_along_axis(tv2, i2, axis=1)
        outk_ref[0, t] = jnp.where(big, gk, jnp.where(one, tk1, zero))
        outv_ref[0, t] = jnp.where(big, gv, jnp.where(one, tv1, zero))


def kernel(inputs, brother_table, relation_type, num_heads):
    del relation_type, num_heads
    tk = brother_table[:, :_D].T
    tv = brother_table[:, _D:].T
    tk2, tv2 = tk[:, 2:], tv[:, 2:]
    tk1, tv1 = tk[:, 1:2], tv[:, 1:2]

    ide = jnp.broadcast_to(
        inputs.reshape(_B, _S, _S, 1), (_B, _S, _S, _H)
    ).reshape(_B * _S * _S // (_TI * _D), _TI, 1, _H * _D)

    grid = (_B * _S * _S // (_TI * _D),)

    def ide_map(g):
        return (g, 0, 0, 0)

    def out_map(g):
        return (g // (_S // _TI), g % (_S // _TI), 0, 0)

    f = pl.pallas_call(
        _tc_body,
        grid=grid,
        in_specs=[
            pl.BlockSpec((_D, 128), lambda g: (0, 0)),
            pl.BlockSpec((_D, 128), lambda g: (0, 0)),
            pl.BlockSpec((_D, 1), lambda g: (0, 0)),
            pl.BlockSpec((_D, 1), lambda g: (0, 0)),
            pl.BlockSpec((1, _TI, 1, _H * _D), ide_map),
        ],
        out_specs=[
            pl.BlockSpec((1, _TI, _D, _H * _D), out_map),
            pl.BlockSpec((1, _TI, _D, _H * _D), out_map),
        ],
        out_shape=[
            jax.ShapeDtypeStruct((_B * _H, _S, _D, _S), jnp.float32),
            jax.ShapeDtypeStruct((_B * _H, _S, _D, _S), jnp.float32),
        ],
    )
    outk, outv = f(tk2, tv2, tk1, tv1, ide)
    return (
        jnp.transpose(outk, (0, 1, 3, 2)),
        jnp.transpose(outv, (0, 1, 3, 2)),
    )
